# baseline (speedup 1.0000x reference)
import functools

import jax
import jax.numpy as jnp
from jax.experimental import pallas as pl
from jax.experimental.pallas import tpu as pltpu

_BATCH = 1024
_OUT = 20
_K = 130107
_BK = 2048

_CONTRACT = (((1,), (1,)), ((), ()))


def _body(x_ref, w_ref, b_ref, o_ref, *, nk, bk, k_total):
    k = pl.program_id(0)

    @pl.when(k == 0)
    def _init():
        o_ref[...] = jnp.zeros_like(o_ref)

    @pl.when(k < nk - 1)
    def _full():
        o_ref[...] += jax.lax.dot_general(
            x_ref[...], w_ref[...], _CONTRACT,
            preferred_element_type=jnp.float32)

    @pl.when(k == nk - 1)
    def _tail():
        valid = k_total - (nk - 1) * bk
        xm = jax.lax.broadcasted_iota(jnp.int32, x_ref.shape, 1) < valid
        wm = jax.lax.broadcasted_iota(jnp.int32, w_ref.shape, 1) < valid
        x = jnp.where(xm, x_ref[...], 0.0)
        w = jnp.where(wm, w_ref[...], 0.0)
        logits = o_ref[...] + jax.lax.dot_general(
            x, w, _CONTRACT, preferred_element_type=jnp.float32)
        logits += b_ref[...]
        m = jnp.max(logits, axis=-1, keepdims=True)
        e = jnp.exp(logits - m)
        o_ref[...] = e / jnp.sum(e, axis=-1, keepdims=True)


def kernel(X, W, b):
    nk = pl.cdiv(_K, _BK)
    body = functools.partial(_body, nk=nk, bk=_BK, k_total=_K)
    return pl.pallas_call(
        body,
        grid=(nk,),
        in_specs=[
            pl.BlockSpec((_BATCH, _BK), lambda k: (0, k)),
            pl.BlockSpec((_OUT, _BK), lambda k: (0, k)),
            pl.BlockSpec((1, _OUT), lambda k: (0, 0)),
        ],
        out_specs=pl.BlockSpec((_BATCH, _OUT), lambda k: (0, 0)),
        out_shape=jax.ShapeDtypeStruct((_BATCH, _OUT), jnp.float32),
        compiler_params=pltpu.CompilerParams(
            dimension_semantics=("arbitrary",)),
    )(X, W, b.reshape(1, _OUT))


# speedup vs baseline: 3.7668x; 3.7668x over previous
import functools

import jax
import jax.numpy as jnp
from jax.experimental import pallas as pl
from jax.experimental.pallas import tpu as pltpu

_BATCH = 1024
_OUT = 20
_K = 130107
_BK = 2048

_CONTRACT = (((1,), (0,)), ((), ()))


def _body(xt_ref, w_ref, b_ref, o_ref, *, nk, bk, k_total):
    k = pl.program_id(0)

    @pl.when(k == 0)
    def _init():
        o_ref[...] = jnp.zeros_like(o_ref)

    @pl.when(k < nk - 1)
    def _full():
        o_ref[...] += jax.lax.dot_general(
            w_ref[...], xt_ref[...], _CONTRACT,
            preferred_element_type=jnp.float32)

    @pl.when(k == nk - 1)
    def _tail():
        valid = k_total - (nk - 1) * bk
        xm = jax.lax.broadcasted_iota(jnp.int32, xt_ref.shape, 0) < valid
        wm = jax.lax.broadcasted_iota(jnp.int32, w_ref.shape, 1) < valid
        x = jnp.where(xm, xt_ref[...], 0.0)
        w = jnp.where(wm, w_ref[...], 0.0)
        logits = o_ref[...] + jax.lax.dot_general(
            w, x, _CONTRACT, preferred_element_type=jnp.float32)
        logits += b_ref[...]
        m = jnp.max(logits, axis=0, keepdims=True)
        e = jnp.exp(logits - m)
        o_ref[...] = e / jnp.sum(e, axis=0, keepdims=True)


def kernel(X, W, b):
    nk = pl.cdiv(_K, _BK)
    body = functools.partial(_body, nk=nk, bk=_BK, k_total=_K)
    out_t = pl.pallas_call(
        body,
        grid=(nk,),
        in_specs=[
            pl.BlockSpec((_BK, _BATCH), lambda k: (k, 0)),
            pl.BlockSpec((_OUT, _BK), lambda k: (0, k)),
            pl.BlockSpec((_OUT, 1), lambda k: (0, 0)),
        ],
        out_specs=pl.BlockSpec((_OUT, _BATCH), lambda k: (0, 0)),
        out_shape=jax.ShapeDtypeStruct((_OUT, _BATCH), jnp.float32),
        compiler_params=pltpu.CompilerParams(
            dimension_semantics=("arbitrary",)),
    )(X.T, W, b.reshape(_OUT, 1))
    return out_t.T
